# baseline (speedup 1.0000x reference)
import functools

import jax
import jax.numpy as jnp
from jax import lax
from jax.experimental import pallas as pl
from jax.experimental.pallas import tpu as pltpu
from jax.experimental.pallas import tpu_sc as plsc

N = 10000
E = 320000
NC = 2
NS = 16
NW = NC * NS
EW = E // NW
CH = 80
NCH = EW // CH
CHW = 128
SLAB = 1
CHA = CHW * SLAB
EWP = 10240
NCHA = EWP // CHA
NBUF = 4
f32 = jnp.float32
i32 = jnp.int32


def _mesh():
    return plsc.VectorSubcoreMesh(core_axis_name="c", subcore_axis_name="s",
                                  num_cores=NC, num_subcores=NS)


@functools.partial(
    pl.kernel,
    out_type=jax.ShapeDtypeStruct((NC * N,), f32),
    mesh=_mesh(),
    compiler_params=pltpu.CompilerParams(needs_layout_passes=False, use_tc_tiling_on_sc=False),
    scratch_types=[pltpu.VMEM((NCH, CH), i32),
                   pltpu.VMEM((NCH, CH), f32),
                   pltpu.VMEM((1000,), f32),
                   pltpu.VMEM_SHARED((N,), f32)],
)
def _sc_deg(dst_h, w_h, z_h, out_h, dst_v, w_v, tmp_v, acc_sh):
    c = lax.axis_index("c")
    s = lax.axis_index("s")
    wid = c * NS + s

    @pl.when(s < 10)
    def _zero():
        sl = pl.ds(s * 1000, 1000)
        pltpu.sync_copy(z_h.at[sl], tmp_v)
        pltpu.sync_copy(tmp_v, acc_sh.at[sl])

    plsc.subcore_barrier()
    pltpu.sync_copy(dst_h.at[wid], dst_v)
    pltpu.sync_copy(w_h.at[wid], w_v)

    def chunk(k, car):
        pltpu.sync_copy(w_v.at[k], acc_sh.at[dst_v.at[k]], add=True)
        return car

    lax.fori_loop(0, NCH, chunk, 0)
    plsc.subcore_barrier()

    @pl.when(s < 10)
    def _wr():
        pltpu.sync_copy(acc_sh.at[pl.ds(s * 1000, 1000)], tmp_v)
        pltpu.sync_copy(tmp_v, out_h.at[pl.ds(c * N + s * 1000, 1000)])


@functools.partial(
    pl.kernel,
    out_type=jax.ShapeDtypeStruct((E,), f32),
    mesh=_mesh(),
    compiler_params=pltpu.CompilerParams(needs_layout_passes=False, use_tc_tiling_on_sc=False),
    scratch_types=[pltpu.VMEM((EW,), i32),
                   pltpu.VMEM((EW,), i32),
                   pltpu.VMEM((EW,), f32),
                   pltpu.VMEM((EW,), f32),
                   pltpu.VMEM((N,), f32)],
)
def _sc_norm(src_h, dst_h, w_h, dinv_h, out_h, src_v, dst_v, w_v, nv, dinv_v):
    c = lax.axis_index("c")
    s = lax.axis_index("s")
    wid = c * NS + s
    sl_e = pl.ds(wid * EW, EW)
    pltpu.sync_copy(src_h.at[sl_e], src_v)
    pltpu.sync_copy(dst_h.at[sl_e], dst_v)
    pltpu.sync_copy(w_h.at[sl_e], w_v)
    pltpu.sync_copy(dinv_h, dinv_v)

    def it(i, car):
        sl = pl.ds(i * 16, 16)
        gs = plsc.load_gather(dinv_v, [src_v[sl]])
        gd = plsc.load_gather(dinv_v, [dst_v[sl]])
        nv[sl] = gs * w_v[sl] * gd
        return car

    lax.fori_loop(0, EW // 16, it, 0)
    pltpu.sync_copy(nv, out_h.at[sl_e])


def _make_agg(D):
    scratch = [pltpu.VMEM((NCHA, CHW), i32),
               pltpu.VMEM((NCHA, CHW), i32),
               pltpu.VMEM((EWP,), f32),
               pltpu.VMEM((1000, D), f32)]
    scratch += [pltpu.VMEM((CHA, D), f32) for _ in range(2 * NBUF)]
    scratch += [pltpu.SemaphoreType.DMA for _ in range(2 * NBUF)]
    scratch += [pltpu.VMEM_SHARED((N, D), f32)]
    spmem_tbl = D <= 16
    if spmem_tbl:
        scratch += [pltpu.VMEM_SHARED((N, D), f32)]

    @functools.partial(
        pl.kernel,
        out_type=jax.ShapeDtypeStruct((NC, N, D), f32),
        mesh=_mesh(),
        compiler_params=pltpu.CompilerParams(needs_layout_passes=False,
                                             use_tc_tiling_on_sc=False),
        scratch_types=scratch,
    )
    def agg(t_h, src_h, dst_h, norm_h, z_h, out_h, *sc):
        src_v, dst_v, norm_v, tmp_v = sc[0], sc[1], sc[2], sc[3]
        gbuf = sc[4:4 + NBUF]
        sbuf = sc[4 + NBUF:4 + 2 * NBUF]
        gsem = sc[4 + 2 * NBUF:4 + 3 * NBUF]
        ssem = sc[4 + 3 * NBUF:4 + 4 * NBUF]
        if spmem_tbl:
            acc_sh, tbl_sh = sc[-2], sc[-1]
        else:
            acc_sh, tbl_sh = sc[-1], t_h
        c = lax.axis_index("c")
        s = lax.axis_index("s")
        wid = c * NS + s

        @pl.when(s < 10)
        def _zero():
            sl = pl.ds(s * 1000, 1000)
            pltpu.sync_copy(z_h.at[sl], tmp_v)
            pltpu.sync_copy(tmp_v, acc_sh.at[sl])
            if spmem_tbl:
                pltpu.sync_copy(t_h.at[sl], tmp_v)
                pltpu.sync_copy(tmp_v, tbl_sh.at[sl])

        plsc.subcore_barrier()
        pltpu.sync_copy(src_h.at[wid], src_v)
        pltpu.sync_copy(dst_h.at[wid], dst_v)
        pltpu.sync_copy(norm_h.at[pl.ds(wid * EWP, EWP)], norm_v)

        for b in range(NBUF):
            pltpu.async_copy(tbl_sh.at[src_v.at[b]], gbuf[b], gsem[b])

        def round_(r, car):
            for b in range(NBUF):
                kk = r * NBUF + b
                pltpu.make_async_copy(
                    tbl_sh.at[src_v.at[kk]], gbuf[b], gsem[b]).wait()

                @pl.when(r > 0)
                def _drain():
                    pltpu.make_async_copy(
                        sbuf[b],
                        acc_sh.at[dst_v.at[lax.max(kk - NBUF, 0)]],
                        ssem[b]).wait()

                def scale(i, car2):
                    for u in range(4):
                        row = i * 4 + u
                        nsp = plsc.load_gather(
                            norm_v,
                            [jnp.full((16,), kk * CHA + row, dtype=i32)])
                        for j in range(D // 16):
                            sl = pl.ds(j * 16, 16)
                            sbuf[b][row, sl] = gbuf[b][row, sl] * nsp
                    return car2

                lax.fori_loop(0, CHA // 4, scale, 0)

                @pl.when(kk + NBUF < NCHA)
                def _regather():
                    pltpu.async_copy(
                        tbl_sh.at[src_v.at[kk + NBUF]], gbuf[b], gsem[b])

                pltpu.async_copy(
                    sbuf[b], acc_sh.at[dst_v.at[kk]], ssem[b], add=True)
            return car

        lax.fori_loop(0, NCHA // NBUF, round_, 0)
        for b in range(NBUF):
            pltpu.make_async_copy(
                sbuf[b], acc_sh.at[dst_v.at[NCHA - NBUF + b]],
                ssem[b]).wait()
        plsc.subcore_barrier()

        @pl.when(s < 10)
        def _wr():
            sl = pl.ds(s * 1000, 1000)
            pltpu.sync_copy(acc_sh.at[sl], tmp_v)
            pltpu.sync_copy(tmp_v, out_h.at[c, sl])

    return agg


_agg16 = _make_agg(16)
_agg32 = _make_agg(32)


@functools.partial(
    pl.kernel,
    out_type=jax.ShapeDtypeStruct((NC * N,), f32),
    mesh=_mesh(),
    compiler_params=pltpu.CompilerParams(needs_layout_passes=False, use_tc_tiling_on_sc=False),
    scratch_types=[pltpu.VMEM((NCH, CH), i32),
                   pltpu.VMEM((NCH, CH), i32),
                   pltpu.VMEM((EW,), f32),
                   pltpu.VMEM((CH,), f32),
                   pltpu.VMEM((1000,), f32),
                   pltpu.VMEM_SHARED((N,), f32),
                   pltpu.VMEM_SHARED((N,), f32)],
)
def _agg1(t_h, src_h, dst_h, norm_h, z_h, out_h,
          src_v, dst_v, norm_v, rows_v, tmp_v, acc_sh, tbl_sh):
    c = lax.axis_index("c")
    s = lax.axis_index("s")
    wid = c * NS + s

    @pl.when(s < 10)
    def _zero():
        sl = pl.ds(s * 1000, 1000)
        pltpu.sync_copy(z_h.at[sl], tmp_v)
        pltpu.sync_copy(tmp_v, acc_sh.at[sl])
        pltpu.sync_copy(t_h.at[sl], tmp_v)
        pltpu.sync_copy(tmp_v, tbl_sh.at[sl])

    plsc.subcore_barrier()
    pltpu.sync_copy(src_h.at[wid], src_v)
    pltpu.sync_copy(dst_h.at[wid], dst_v)
    pltpu.sync_copy(norm_h.at[pl.ds(wid * EW, EW)], norm_v)

    def chunk(k, car):
        pltpu.sync_copy(tbl_sh.at[src_v.at[k]], rows_v)
        for j in range(CH // 16):
            sl = pl.ds(j * 16, 16)
            rows_v[sl] = rows_v[sl] * norm_v[pl.ds(k * CH + j * 16, 16)]
        pltpu.sync_copy(rows_v, acc_sh.at[dst_v.at[k]], add=True)
        return car

    lax.fori_loop(0, NCH, chunk, 0)
    plsc.subcore_barrier()

    @pl.when(s < 10)
    def _wr():
        pltpu.sync_copy(acc_sh.at[pl.ds(s * 1000, 1000)], tmp_v)
        pltpu.sync_copy(tmp_v, out_h.at[pl.ds(c * N + s * 1000, 1000)])


def _tc(body, out_shapes):
    return pl.pallas_call(
        body,
        out_shape=[jax.ShapeDtypeStruct(s, f32) for s in out_shapes])


def _tc_first(deg2t, x, w1):
    def body(deg_ref, x_ref, w_ref, dinv_ref, sn_ref, t_ref):
        deg = deg_ref[:, 0:1] + deg_ref[:, 1:2] + 1.0
        di = lax.rsqrt(deg)
        dinv_ref[...] = di
        sn_ref[...] = di * di
        t_ref[...] = jnp.dot(x_ref[...], w_ref[...],
                             preferred_element_type=f32)
    return _tc(body, [(N, 1), (N, 1), (N, w1.shape[1])])(deg2t, x, w1)


def _tc_combine(a, sn, t, b):
    def body(a_ref, sn_ref, t_ref, b_ref, o_ref):
        h = a_ref[0] + a_ref[1] + sn_ref[...] * t_ref[...] + b_ref[...]
        o_ref[...] = jnp.maximum(h, 0.0)
    return _tc(body, [t.shape])(a, sn, t, b[None, :])[0]


def _tc_aggmm(a, sn, h, w, b):
    def body(a_ref, sn_ref, h_ref, w_ref, b_ref, o_ref):
        g = a_ref[0] + a_ref[1] + sn_ref[...] * h_ref[...]
        o_ref[...] = jnp.maximum(
            jnp.dot(g, w_ref[...], preferred_element_type=f32) + b_ref[...],
            0.0)
    return _tc(body, [(N, w.shape[1])])(a, sn, h, w, b[None, :])[0]


def _tc_aggmm2(a, sn, h, w, b, w2):
    def body(a_ref, sn_ref, h_ref, w_ref, b_ref, w2_ref, o_ref):
        g = a_ref[0] + a_ref[1] + sn_ref[...] * h_ref[...]
        hn = jnp.maximum(
            jnp.dot(g, w_ref[...], preferred_element_type=f32) + b_ref[...],
            0.0)
        o_ref[...] = jnp.dot(hn, w2_ref[...], preferred_element_type=f32)
    return _tc(body, [(N, w2.shape[1])])(a, sn, h, w, b[None, :], w2)[0]


def _tc_aggmm2_cat(aa, ab, sn, h, w, b, w2):
    def body(aa_ref, ab_ref, sn_ref, h_ref, w_ref, b_ref, w2_ref, o_ref):
        g = jnp.concatenate(
            [aa_ref[0] + aa_ref[1], ab_ref[0] + ab_ref[1]], axis=1)
        g = g + sn_ref[...] * h_ref[...]
        hn = jnp.maximum(
            jnp.dot(g, w_ref[...], preferred_element_type=f32) + b_ref[...],
            0.0)
        o_ref[...] = jnp.dot(hn, w2_ref[...], preferred_element_type=f32)
    return _tc(body, [(N, w2.shape[1])])(aa, ab, sn, h, w, b[None, :], w2)[0]


def _tc_final(a10t, sn, t10, b10):
    def body(a_ref, sn_ref, t_ref, b_ref, o_ref):
        o_ref[...] = jax.nn.sigmoid(
            a_ref[:, 0:1] + a_ref[:, 1:2]
            + sn_ref[...] * t_ref[...] + b_ref[...])
    return _tc(body, [(N, 1)])(a10t, sn, t10, b10[None, :])[0]


def kernel(x, edge_index, edge_weight, W1, b1, W2, b2, W3, b3, W4, b4, W5, b5,
           W6, b6, W7, b7, W8, b8, W9, b9, W10, b10):
    src = edge_index[0]
    dst = edge_index[1]
    src3 = src.reshape(NW, NCH, CH)
    dst3 = dst.reshape(NW, NCH, CH)
    pad2 = ((0, 0), (0, EWP - EW))
    srcA = jnp.pad(src.reshape(NW, EW), pad2).reshape(NW, NCHA, CHW)
    dstA = jnp.pad(dst.reshape(NW, EW), pad2).reshape(NW, NCHA, CHW)
    w3 = edge_weight.reshape(NW, NCH, CH)

    z1 = jnp.zeros((N,), f32)
    zD = {d: jnp.zeros((N, d), f32) for d in (16, 32)}

    deg2 = _sc_deg(dst3, w3, z1).reshape(NC, N)
    dinv, sn, t1 = _tc_first(deg2.T, x, W1)
    normf = _sc_norm(src, dst, edge_weight, dinv.reshape(N))

    normP = jnp.pad(normf.reshape(NW, EW), pad2).reshape(NW * EWP)

    def agg(t, d):
        k = {16: _agg16, 32: _agg32}[d]
        return k(t, srcA, dstA, normP, zD[d])

    a1 = agg(t1, 16)
    h2 = _tc_combine(a1, sn, t1, b1)
    a2 = agg(h2, 16)
    h3 = _tc_aggmm(a2, sn, h2, W2, b2)
    a3 = agg(h3, 32)
    h4 = _tc_aggmm(a3, sn, h3, W3, b3)
    a4a = agg(h4[:, :32], 32)
    a4b = agg(h4[:, 32:], 32)
    t5 = _tc_aggmm2_cat(a4a, a4b, sn, h4, W4, b4, W5)
    a5 = agg(t5, 32)
    h6 = _tc_combine(a5, sn, t5, b5)
    a6 = agg(h6, 32)
    t7 = _tc_aggmm2(a6, sn, h6, W6, b6, W7)
    a7 = agg(t7, 16)
    h8 = _tc_combine(a7, sn, t7, b7)
    a8 = agg(h8, 16)
    h9 = _tc_aggmm(a8, sn, h8, W8, b8)
    a9 = agg(h9, 16)
    t10 = _tc_aggmm2(a9, sn, h9, W9, b9, W10)
    a10 = _agg1(t10.reshape(N), src3, dst3, normf, z1).reshape(NC, N)
    out = _tc_final(a10.T, sn, t10, b10)
    return out.reshape(N)


# speedup vs baseline: 1.0105x; 1.0105x over previous
import functools

import jax
import jax.numpy as jnp
from jax import lax
from jax.experimental import pallas as pl
from jax.experimental.pallas import tpu as pltpu
from jax.experimental.pallas import tpu_sc as plsc

N = 10000
E = 320000
NC = 2
NS = 16
NW = NC * NS
EW = E // NW
CH = 80
NCH = EW // CH
CHW = 128
SLAB = 1
CHA = CHW * SLAB
EWP = 10240
NCHA = EWP // CHA
NBUF = 4
f32 = jnp.float32
i32 = jnp.int32


def _mesh():
    return plsc.VectorSubcoreMesh(core_axis_name="c", subcore_axis_name="s",
                                  num_cores=NC, num_subcores=NS)


@functools.partial(
    pl.kernel,
    out_type=jax.ShapeDtypeStruct((NC * N,), f32),
    mesh=_mesh(),
    compiler_params=pltpu.CompilerParams(needs_layout_passes=False, use_tc_tiling_on_sc=False),
    scratch_types=[pltpu.VMEM((NCH, CH), i32),
                   pltpu.VMEM((NCH, CH), f32),
                   pltpu.VMEM((1000,), f32),
                   pltpu.VMEM_SHARED((N,), f32)],
)
def _sc_deg(dst_h, w_h, z_h, out_h, dst_v, w_v, tmp_v, acc_sh):
    c = lax.axis_index("c")
    s = lax.axis_index("s")
    wid = c * NS + s

    @pl.when(s < 10)
    def _zero():
        sl = pl.ds(s * 1000, 1000)
        pltpu.sync_copy(z_h.at[sl], tmp_v)
        pltpu.sync_copy(tmp_v, acc_sh.at[sl])

    plsc.subcore_barrier()
    pltpu.sync_copy(dst_h.at[wid], dst_v)
    pltpu.sync_copy(w_h.at[wid], w_v)

    def chunk(k, car):
        pltpu.sync_copy(w_v.at[k], acc_sh.at[dst_v.at[k]], add=True)
        return car

    lax.fori_loop(0, NCH, chunk, 0)
    plsc.subcore_barrier()

    @pl.when(s < 10)
    def _wr():
        pltpu.sync_copy(acc_sh.at[pl.ds(s * 1000, 1000)], tmp_v)
        pltpu.sync_copy(tmp_v, out_h.at[pl.ds(c * N + s * 1000, 1000)])


@functools.partial(
    pl.kernel,
    out_type=jax.ShapeDtypeStruct((E,), f32),
    mesh=_mesh(),
    compiler_params=pltpu.CompilerParams(needs_layout_passes=False, use_tc_tiling_on_sc=False),
    scratch_types=[pltpu.VMEM((EW,), i32),
                   pltpu.VMEM((EW,), i32),
                   pltpu.VMEM((EW,), f32),
                   pltpu.VMEM((EW,), f32),
                   pltpu.VMEM((N,), f32)],
)
def _sc_norm(src_h, dst_h, w_h, dinv_h, out_h, src_v, dst_v, w_v, nv, dinv_v):
    c = lax.axis_index("c")
    s = lax.axis_index("s")
    wid = c * NS + s
    sl_e = pl.ds(wid * EW, EW)
    pltpu.sync_copy(src_h.at[sl_e], src_v)
    pltpu.sync_copy(dst_h.at[sl_e], dst_v)
    pltpu.sync_copy(w_h.at[sl_e], w_v)
    pltpu.sync_copy(dinv_h, dinv_v)

    def it(i, car):
        sl = pl.ds(i * 16, 16)
        gs = plsc.load_gather(dinv_v, [src_v[sl]])
        gd = plsc.load_gather(dinv_v, [dst_v[sl]])
        nv[sl] = gs * w_v[sl] * gd
        return car

    lax.fori_loop(0, EW // 16, it, 0)
    pltpu.sync_copy(nv, out_h.at[sl_e])


def _make_agg(D):
    scratch = [pltpu.VMEM((NCHA, CHW), i32),
               pltpu.VMEM((NCHA, CHW), i32),
               pltpu.VMEM((EWP,), f32),
               pltpu.VMEM((1000, D), f32)]
    scratch += [pltpu.VMEM((CHA, D), f32) for _ in range(2 * NBUF)]
    scratch += [pltpu.SemaphoreType.DMA for _ in range(2 * NBUF)]
    scratch += [pltpu.VMEM_SHARED((N, D), f32)]
    spmem_tbl = D <= 16
    if spmem_tbl:
        scratch += [pltpu.VMEM_SHARED((N, D), f32)]

    @functools.partial(
        pl.kernel,
        out_type=jax.ShapeDtypeStruct((NC, N, D), f32),
        mesh=_mesh(),
        compiler_params=pltpu.CompilerParams(needs_layout_passes=False,
                                             use_tc_tiling_on_sc=False),
        scratch_types=scratch,
    )
    def agg(t_h, src_h, dst_h, norm_h, z_h, out_h, *sc):
        src_v, dst_v, norm_v, tmp_v = sc[0], sc[1], sc[2], sc[3]
        gbuf = sc[4:4 + NBUF]
        sbuf = sc[4 + NBUF:4 + 2 * NBUF]
        gsem = sc[4 + 2 * NBUF:4 + 3 * NBUF]
        ssem = sc[4 + 3 * NBUF:4 + 4 * NBUF]
        if spmem_tbl:
            acc_sh, tbl_sh = sc[-2], sc[-1]
        else:
            acc_sh, tbl_sh = sc[-1], t_h
        c = lax.axis_index("c")
        s = lax.axis_index("s")
        wid = c * NS + s

        @pl.when(s < 10)
        def _zero():
            sl = pl.ds(s * 1000, 1000)
            pltpu.sync_copy(z_h.at[sl], tmp_v)
            pltpu.sync_copy(tmp_v, acc_sh.at[sl])
            if spmem_tbl:
                pltpu.sync_copy(t_h.at[sl], tmp_v)
                pltpu.sync_copy(tmp_v, tbl_sh.at[sl])

        plsc.subcore_barrier()
        pltpu.sync_copy(src_h.at[wid], src_v)
        pltpu.sync_copy(dst_h.at[wid], dst_v)
        pltpu.sync_copy(norm_h.at[pl.ds(wid * EWP, EWP)], norm_v)

        for b in range(NBUF):
            pltpu.async_copy(tbl_sh.at[src_v.at[b]], gbuf[b], gsem[b])

        def round_(r, car):
            for b in range(NBUF):
                kk = r * NBUF + b
                pltpu.make_async_copy(
                    tbl_sh.at[src_v.at[kk]], gbuf[b], gsem[b]).wait()

                @pl.when(r > 0)
                def _drain():
                    pltpu.make_async_copy(
                        sbuf[b],
                        acc_sh.at[dst_v.at[lax.max(kk - NBUF, 0)]],
                        ssem[b]).wait()

                def scale(i, car2):
                    for u in range(4):
                        row = i * 4 + u
                        nsp = plsc.load_gather(
                            norm_v,
                            [jnp.full((16,), kk * CHA + row, dtype=i32)])
                        for j in range(D // 16):
                            sl = pl.ds(j * 16, 16)
                            sbuf[b][row, sl] = gbuf[b][row, sl] * nsp
                    return car2

                lax.fori_loop(0, CHA // 4, scale, 0)

                @pl.when(kk + NBUF < NCHA)
                def _regather():
                    pltpu.async_copy(
                        tbl_sh.at[src_v.at[kk + NBUF]], gbuf[b], gsem[b])

                pltpu.async_copy(
                    sbuf[b], acc_sh.at[dst_v.at[kk]], ssem[b], add=True)
            return car

        lax.fori_loop(0, NCHA // NBUF, round_, 0)
        for b in range(NBUF):
            pltpu.make_async_copy(
                sbuf[b], acc_sh.at[dst_v.at[NCHA - NBUF + b]],
                ssem[b]).wait()
        plsc.subcore_barrier()

        @pl.when(s < 10)
        def _wr():
            sl = pl.ds(s * 1000, 1000)
            pltpu.sync_copy(acc_sh.at[sl], tmp_v)
            pltpu.sync_copy(tmp_v, out_h.at[c, sl])

    return agg


_agg16 = _make_agg(16)
_agg32 = _make_agg(32)


@functools.partial(
    pl.kernel,
    out_type=jax.ShapeDtypeStruct((NC * N,), f32),
    mesh=_mesh(),
    compiler_params=pltpu.CompilerParams(needs_layout_passes=False, use_tc_tiling_on_sc=False),
    scratch_types=[pltpu.VMEM((NCH, CH), i32),
                   pltpu.VMEM((NCH, CH), i32),
                   pltpu.VMEM((EW,), f32),
                   pltpu.VMEM((CH,), f32),
                   pltpu.VMEM((1000,), f32),
                   pltpu.VMEM_SHARED((N,), f32),
                   pltpu.VMEM_SHARED((N,), f32)],
)
def _agg1(t_h, src_h, dst_h, norm_h, z_h, out_h,
          src_v, dst_v, norm_v, rows_v, tmp_v, acc_sh, tbl_sh):
    c = lax.axis_index("c")
    s = lax.axis_index("s")
    wid = c * NS + s

    @pl.when(s < 10)
    def _zero():
        sl = pl.ds(s * 1000, 1000)
        pltpu.sync_copy(z_h.at[sl], tmp_v)
        pltpu.sync_copy(tmp_v, acc_sh.at[sl])
        pltpu.sync_copy(t_h.at[sl], tmp_v)
        pltpu.sync_copy(tmp_v, tbl_sh.at[sl])

    plsc.subcore_barrier()
    pltpu.sync_copy(src_h.at[wid], src_v)
    pltpu.sync_copy(dst_h.at[wid], dst_v)
    pltpu.sync_copy(norm_h.at[pl.ds(wid * EW, EW)], norm_v)

    def chunk(k, car):
        pltpu.sync_copy(tbl_sh.at[src_v.at[k]], rows_v)
        for j in range(CH // 16):
            sl = pl.ds(j * 16, 16)
            rows_v[sl] = rows_v[sl] * norm_v[pl.ds(k * CH + j * 16, 16)]
        pltpu.sync_copy(rows_v, acc_sh.at[dst_v.at[k]], add=True)
        return car

    lax.fori_loop(0, NCH, chunk, 0)
    plsc.subcore_barrier()

    @pl.when(s < 10)
    def _wr():
        pltpu.sync_copy(acc_sh.at[pl.ds(s * 1000, 1000)], tmp_v)
        pltpu.sync_copy(tmp_v, out_h.at[pl.ds(c * N + s * 1000, 1000)])


def _tc(body, out_shapes):
    return pl.pallas_call(
        body,
        out_shape=[jax.ShapeDtypeStruct(s, f32) for s in out_shapes])


def _tc_first(deg2t, x, w1):
    def body(deg_ref, x_ref, w_ref, dinv_ref, sn_ref, t_ref):
        deg = deg_ref[:, 0:1] + deg_ref[:, 1:2] + 1.0
        di = lax.rsqrt(deg)
        dinv_ref[...] = di
        sn_ref[...] = di * di
        t_ref[...] = jnp.dot(x_ref[...], w_ref[...],
                             preferred_element_type=f32)
    return _tc(body, [(N, 1), (N, 1), (N, w1.shape[1])])(deg2t, x, w1)


def _tc_combine(a, sn, t, b):
    def body(a_ref, sn_ref, t_ref, b_ref, o_ref):
        h = a_ref[0] + a_ref[1] + sn_ref[...] * t_ref[...] + b_ref[...]
        o_ref[...] = jnp.maximum(h, 0.0)
    return _tc(body, [t.shape])(a, sn, t, b[None, :])[0]


def _tc_aggmm(a, sn, h, w, b):
    def body(a_ref, sn_ref, h_ref, w_ref, b_ref, o_ref):
        g = a_ref[0] + a_ref[1] + sn_ref[...] * h_ref[...]
        o_ref[...] = jnp.maximum(
            jnp.dot(g, w_ref[...], preferred_element_type=f32) + b_ref[...],
            0.0)
    return _tc(body, [(N, w.shape[1])])(a, sn, h, w, b[None, :])[0]


def _tc_aggmm2(a, sn, h, w, b, w2):
    def body(a_ref, sn_ref, h_ref, w_ref, b_ref, w2_ref, o_ref):
        g = a_ref[0] + a_ref[1] + sn_ref[...] * h_ref[...]
        hn = jnp.maximum(
            jnp.dot(g, w_ref[...], preferred_element_type=f32) + b_ref[...],
            0.0)
        o_ref[...] = jnp.dot(hn, w2_ref[...], preferred_element_type=f32)
    return _tc(body, [(N, w2.shape[1])])(a, sn, h, w, b[None, :], w2)[0]


def _tc_aggmm2_cat(aa, ab, sn, h, w, b, w2):
    def body(aa_ref, ab_ref, sn_ref, h_ref, w_ref, b_ref, w2_ref, o_ref):
        g = jnp.concatenate(
            [aa_ref[0] + aa_ref[1], ab_ref[0] + ab_ref[1]], axis=1)
        g = g + sn_ref[...] * h_ref[...]
        hn = jnp.maximum(
            jnp.dot(g, w_ref[...], preferred_element_type=f32) + b_ref[...],
            0.0)
        o_ref[...] = jnp.dot(hn, w2_ref[...], preferred_element_type=f32)
    return _tc(body, [(N, w2.shape[1])])(aa, ab, sn, h, w, b[None, :], w2)[0]


def _tc_final(a10t, sn, t10, b10):
    def body(a_ref, sn_ref, t_ref, b_ref, o_ref):
        o_ref[...] = jax.nn.sigmoid(
            a_ref[:, 0:1] + a_ref[:, 1:2]
            + sn_ref[...] * t_ref[...] + b_ref[...])
    return _tc(body, [(N, 1)])(a10t, sn, t10, b10[None, :])[0]


def kernel(x, edge_index, edge_weight, W1, b1, W2, b2, W3, b3, W4, b4, W5, b5,
           W6, b6, W7, b7, W8, b8, W9, b9, W10, b10):
    src = edge_index[0]
    dst = edge_index[1]
    src3 = src.reshape(NW, NCH, CH)
    dst3 = dst.reshape(NW, NCH, CH)
    pad2 = ((0, 0), (0, EWP - EW))
    dpad = jnp.broadcast_to((jnp.arange(EWP - EW, dtype=i32) * 37) % N,
                            (NW, EWP - EW))
    srcA = jnp.pad(src.reshape(NW, EW), pad2).reshape(NW, NCHA, CHW)
    dstA = jnp.concatenate([dst.reshape(NW, EW), dpad],
                           axis=1).reshape(NW, NCHA, CHW)
    w3 = edge_weight.reshape(NW, NCH, CH)

    z1 = jnp.zeros((N,), f32)
    zD = {d: jnp.zeros((N, d), f32) for d in (16, 32)}

    deg2 = _sc_deg(dst3, w3, z1).reshape(NC, N)
    dinv, sn, t1 = _tc_first(deg2.T, x, W1)
    normf = _sc_norm(src, dst, edge_weight, dinv.reshape(N))

    normP = jnp.pad(normf.reshape(NW, EW), pad2).reshape(NW * EWP)

    def agg(t, d):
        k = {16: _agg16, 32: _agg32}[d]
        return k(t, srcA, dstA, normP, zD[d])

    a1 = agg(t1, 16)
    h2 = _tc_combine(a1, sn, t1, b1)
    a2 = agg(h2, 16)
    h3 = _tc_aggmm(a2, sn, h2, W2, b2)
    a3 = agg(h3, 32)
    h4 = _tc_aggmm(a3, sn, h3, W3, b3)
    a4a = agg(h4[:, :32], 32)
    a4b = agg(h4[:, 32:], 32)
    t5 = _tc_aggmm2_cat(a4a, a4b, sn, h4, W4, b4, W5)
    a5 = agg(t5, 32)
    h6 = _tc_combine(a5, sn, t5, b5)
    a6 = agg(h6, 32)
    t7 = _tc_aggmm2(a6, sn, h6, W6, b6, W7)
    a7 = agg(t7, 16)
    h8 = _tc_combine(a7, sn, t7, b7)
    a8 = agg(h8, 16)
    h9 = _tc_aggmm(a8, sn, h8, W8, b8)
    a9 = agg(h9, 16)
    t10 = _tc_aggmm2(a9, sn, h9, W9, b9, W10)
    a10 = _agg1(t10.reshape(N), src3, dst3, normf, z1).reshape(NC, N)
    out = _tc_final(a10.T, sn, t10, b10)
    return out.reshape(N)


# speedup vs baseline: 1.5738x; 1.5575x over previous
import functools

import jax
import jax.numpy as jnp
from jax import lax
from jax.experimental import pallas as pl
from jax.experimental.pallas import tpu as pltpu
from jax.experimental.pallas import tpu_sc as plsc

N = 10000
E = 320000
NC = 2
NS = 16
NW = NC * NS
EW = E // NW
CH = 80
NCH = EW // CH
CHA = 125
NCHA = EW // CHA
NBUF = 4
f32 = jnp.float32
i32 = jnp.int32


def _mesh():
    return plsc.VectorSubcoreMesh(core_axis_name="c", subcore_axis_name="s",
                                  num_cores=NC, num_subcores=NS)


@functools.partial(
    pl.kernel,
    out_type=jax.ShapeDtypeStruct((NC * N,), f32),
    mesh=_mesh(),
    compiler_params=pltpu.CompilerParams(needs_layout_passes=False, use_tc_tiling_on_sc=False),
    scratch_types=[pltpu.VMEM((NCH, CH), i32),
                   pltpu.VMEM((NCH, CH), f32),
                   pltpu.VMEM((1000,), f32),
                   pltpu.VMEM_SHARED((N,), f32)],
)
def _sc_deg(dst_h, w_h, z_h, out_h, dst_v, w_v, tmp_v, acc_sh):
    c = lax.axis_index("c")
    s = lax.axis_index("s")
    wid = c * NS + s

    @pl.when(s < 10)
    def _zero():
        sl = pl.ds(s * 1000, 1000)
        pltpu.sync_copy(z_h.at[sl], tmp_v)
        pltpu.sync_copy(tmp_v, acc_sh.at[sl])

    plsc.subcore_barrier()
    pltpu.sync_copy(dst_h.at[wid], dst_v)
    pltpu.sync_copy(w_h.at[wid], w_v)

    def chunk(k, car):
        pltpu.sync_copy(w_v.at[k], acc_sh.at[dst_v.at[k]], add=True)
        return car

    lax.fori_loop(0, NCH, chunk, 0)
    plsc.subcore_barrier()

    @pl.when(s < 10)
    def _wr():
        pltpu.sync_copy(acc_sh.at[pl.ds(s * 1000, 1000)], tmp_v)
        pltpu.sync_copy(tmp_v, out_h.at[pl.ds(c * N + s * 1000, 1000)])


@functools.partial(
    pl.kernel,
    out_type=jax.ShapeDtypeStruct((E,), f32),
    mesh=_mesh(),
    compiler_params=pltpu.CompilerParams(needs_layout_passes=False, use_tc_tiling_on_sc=False),
    scratch_types=[pltpu.VMEM((EW,), i32),
                   pltpu.VMEM((EW,), i32),
                   pltpu.VMEM((EW,), f32),
                   pltpu.VMEM((EW,), f32),
                   pltpu.VMEM((N,), f32)],
)
def _sc_norm(src_h, dst_h, w_h, dinv_h, out_h, src_v, dst_v, w_v, nv, dinv_v):
    c = lax.axis_index("c")
    s = lax.axis_index("s")
    wid = c * NS + s
    sl_e = pl.ds(wid * EW, EW)
    pltpu.sync_copy(src_h.at[sl_e], src_v)
    pltpu.sync_copy(dst_h.at[sl_e], dst_v)
    pltpu.sync_copy(w_h.at[sl_e], w_v)
    pltpu.sync_copy(dinv_h, dinv_v)

    def it(i, car):
        sl = pl.ds(i * 16, 16)
        gs = plsc.load_gather(dinv_v, [src_v[sl]])
        gd = plsc.load_gather(dinv_v, [dst_v[sl]])
        nv[sl] = gs * w_v[sl] * gd
        return car

    lax.fori_loop(0, EW // 16, it, 0)
    pltpu.sync_copy(nv, out_h.at[sl_e])


def _make_agg(D):
    scratch = [pltpu.VMEM((NCHA, CHA), i32),
               pltpu.VMEM((NCHA, CHA), i32),
               pltpu.VMEM((EW,), f32),
               pltpu.VMEM((1000, D), f32)]
    scratch += [pltpu.VMEM((CHA, D), f32) for _ in range(2 * NBUF)]
    scratch += [pltpu.SemaphoreType.DMA for _ in range(2 * NBUF)]
    scratch += [pltpu.VMEM_SHARED((N, D), f32)]
    spmem_tbl = D <= 16
    if spmem_tbl:
        scratch += [pltpu.VMEM_SHARED((N, D), f32)]

    @functools.partial(
        pl.kernel,
        out_type=jax.ShapeDtypeStruct((NC, N, D), f32),
        mesh=_mesh(),
        compiler_params=pltpu.CompilerParams(needs_layout_passes=False,
                                             use_tc_tiling_on_sc=False),
        scratch_types=scratch,
    )
    def agg(t_h, src_h, dst_h, norm_h, z_h, out_h, *sc):
        src_v, dst_v, norm_v, tmp_v = sc[0], sc[1], sc[2], sc[3]
        gbuf = sc[4:4 + NBUF]
        sbuf = sc[4 + NBUF:4 + 2 * NBUF]
        gsem = sc[4 + 2 * NBUF:4 + 3 * NBUF]
        ssem = sc[4 + 3 * NBUF:4 + 4 * NBUF]
        if spmem_tbl:
            acc_sh, tbl_sh = sc[-2], sc[-1]
        else:
            acc_sh, tbl_sh = sc[-1], t_h
        c = lax.axis_index("c")
        s = lax.axis_index("s")
        wid = c * NS + s

        @pl.when(s < 10)
        def _zero():
            sl = pl.ds(s * 1000, 1000)
            pltpu.sync_copy(z_h.at[sl], tmp_v)
            pltpu.sync_copy(tmp_v, acc_sh.at[sl])
            if spmem_tbl:
                pltpu.sync_copy(t_h.at[sl], tmp_v)
                pltpu.sync_copy(tmp_v, tbl_sh.at[sl])

        plsc.subcore_barrier()
        pltpu.sync_copy(src_h.at[wid], src_v)
        pltpu.sync_copy(dst_h.at[wid], dst_v)
        pltpu.sync_copy(norm_h.at[pl.ds(wid * EW, EW)], norm_v)

        for b in range(NBUF):
            pltpu.async_copy(tbl_sh.at[src_v.at[b]], gbuf[b], gsem[b])

        def round_(r, car):
            for b in range(NBUF):
                kk = r * NBUF + b
                pltpu.make_async_copy(
                    tbl_sh.at[src_v.at[kk]], gbuf[b], gsem[b]).wait()

                @pl.when(r > 0)
                def _drain():
                    pltpu.make_async_copy(
                        sbuf[b],
                        acc_sh.at[dst_v.at[lax.max(kk - NBUF, 0)]],
                        ssem[b]).wait()

                one = jnp.ones((16,), dtype=i32)

                def scale(i, idxv):
                    for u in range(5):
                        row = i * 5 + u
                        nsp = plsc.load_gather(norm_v, [idxv])
                        idxv = idxv + one
                        for j in range(D // 16):
                            sl = pl.ds(j * 16, 16)
                            sbuf[b][row, sl] = gbuf[b][row, sl] * nsp
                    return idxv

                lax.fori_loop(0, CHA // 5, scale,
                              jnp.full((16,), kk * CHA, dtype=i32))

                @pl.when(kk + NBUF < NCHA)
                def _regather():
                    pltpu.async_copy(
                        tbl_sh.at[src_v.at[kk + NBUF]], gbuf[b], gsem[b])

                pltpu.async_copy(
                    sbuf[b], acc_sh.at[dst_v.at[kk]], ssem[b], add=True)
            return car

        lax.fori_loop(0, NCHA // NBUF, round_, 0)
        for b in range(NBUF):
            pltpu.make_async_copy(
                sbuf[b], acc_sh.at[dst_v.at[NCHA - NBUF + b]],
                ssem[b]).wait()
        plsc.subcore_barrier()

        @pl.when(s < 10)
        def _wr():
            sl = pl.ds(s * 1000, 1000)
            pltpu.sync_copy(acc_sh.at[sl], tmp_v)
            pltpu.sync_copy(tmp_v, out_h.at[c, sl])

    return agg


_agg16 = _make_agg(16)
_agg32 = _make_agg(32)


@functools.partial(
    pl.kernel,
    out_type=jax.ShapeDtypeStruct((NC * N,), f32),
    mesh=_mesh(),
    compiler_params=pltpu.CompilerParams(needs_layout_passes=False, use_tc_tiling_on_sc=False),
    scratch_types=[pltpu.VMEM((NCH, CH), i32),
                   pltpu.VMEM((NCH, CH), i32),
                   pltpu.VMEM((EW,), f32),
                   pltpu.VMEM((CH,), f32),
                   pltpu.VMEM((1000,), f32),
                   pltpu.VMEM_SHARED((N,), f32),
                   pltpu.VMEM_SHARED((N,), f32)],
)
def _agg1(t_h, src_h, dst_h, norm_h, z_h, out_h,
          src_v, dst_v, norm_v, rows_v, tmp_v, acc_sh, tbl_sh):
    c = lax.axis_index("c")
    s = lax.axis_index("s")
    wid = c * NS + s

    @pl.when(s < 10)
    def _zero():
        sl = pl.ds(s * 1000, 1000)
        pltpu.sync_copy(z_h.at[sl], tmp_v)
        pltpu.sync_copy(tmp_v, acc_sh.at[sl])
        pltpu.sync_copy(t_h.at[sl], tmp_v)
        pltpu.sync_copy(tmp_v, tbl_sh.at[sl])

    plsc.subcore_barrier()
    pltpu.sync_copy(src_h.at[wid], src_v)
    pltpu.sync_copy(dst_h.at[wid], dst_v)
    pltpu.sync_copy(norm_h.at[pl.ds(wid * EW, EW)], norm_v)

    def chunk(k, car):
        pltpu.sync_copy(tbl_sh.at[src_v.at[k]], rows_v)
        for j in range(CH // 16):
            sl = pl.ds(j * 16, 16)
            rows_v[sl] = rows_v[sl] * norm_v[pl.ds(k * CH + j * 16, 16)]
        pltpu.sync_copy(rows_v, acc_sh.at[dst_v.at[k]], add=True)
        return car

    lax.fori_loop(0, NCH, chunk, 0)
    plsc.subcore_barrier()

    @pl.when(s < 10)
    def _wr():
        pltpu.sync_copy(acc_sh.at[pl.ds(s * 1000, 1000)], tmp_v)
        pltpu.sync_copy(tmp_v, out_h.at[pl.ds(c * N + s * 1000, 1000)])


def _tc(body, out_shapes):
    return pl.pallas_call(
        body,
        out_shape=[jax.ShapeDtypeStruct(s, f32) for s in out_shapes])


def _tc_first(deg2t, x, w1):
    def body(deg_ref, x_ref, w_ref, dinv_ref, sn_ref, t_ref):
        deg = deg_ref[:, 0:1] + deg_ref[:, 1:2] + 1.0
        di = lax.rsqrt(deg)
        dinv_ref[...] = di
        sn_ref[...] = di * di
        t_ref[...] = jnp.dot(x_ref[...], w_ref[...],
                             preferred_element_type=f32)
    return _tc(body, [(N, 1), (N, 1), (N, w1.shape[1])])(deg2t, x, w1)


def _tc_combine(a, sn, t, b):
    def body(a_ref, sn_ref, t_ref, b_ref, o_ref):
        h = a_ref[0] + a_ref[1] + sn_ref[...] * t_ref[...] + b_ref[...]
        o_ref[...] = jnp.maximum(h, 0.0)
    return _tc(body, [t.shape])(a, sn, t, b[None, :])[0]


def _tc_aggmm(a, sn, h, w, b):
    def body(a_ref, sn_ref, h_ref, w_ref, b_ref, o_ref):
        g = a_ref[0] + a_ref[1] + sn_ref[...] * h_ref[...]
        o_ref[...] = jnp.maximum(
            jnp.dot(g, w_ref[...], preferred_element_type=f32) + b_ref[...],
            0.0)
    return _tc(body, [(N, w.shape[1])])(a, sn, h, w, b[None, :])[0]


def _tc_aggmm2(a, sn, h, w, b, w2):
    def body(a_ref, sn_ref, h_ref, w_ref, b_ref, w2_ref, o_ref):
        g = a_ref[0] + a_ref[1] + sn_ref[...] * h_ref[...]
        hn = jnp.maximum(
            jnp.dot(g, w_ref[...], preferred_element_type=f32) + b_ref[...],
            0.0)
        o_ref[...] = jnp.dot(hn, w2_ref[...], preferred_element_type=f32)
    return _tc(body, [(N, w2.shape[1])])(a, sn, h, w, b[None, :], w2)[0]


def _tc_aggmm2_cat(aa, ab, sn, h, w, b, w2):
    def body(aa_ref, ab_ref, sn_ref, h_ref, w_ref, b_ref, w2_ref, o_ref):
        g = jnp.concatenate(
            [aa_ref[0] + aa_ref[1], ab_ref[0] + ab_ref[1]], axis=1)
        g = g + sn_ref[...] * h_ref[...]
        hn = jnp.maximum(
            jnp.dot(g, w_ref[...], preferred_element_type=f32) + b_ref[...],
            0.0)
        o_ref[...] = jnp.dot(hn, w2_ref[...], preferred_element_type=f32)
    return _tc(body, [(N, w2.shape[1])])(aa, ab, sn, h, w, b[None, :], w2)[0]


def _tc_final(a10t, sn, t10, b10):
    def body(a_ref, sn_ref, t_ref, b_ref, o_ref):
        o_ref[...] = jax.nn.sigmoid(
            a_ref[:, 0:1] + a_ref[:, 1:2]
            + sn_ref[...] * t_ref[...] + b_ref[...])
    return _tc(body, [(N, 1)])(a10t, sn, t10, b10[None, :])[0]


def kernel(x, edge_index, edge_weight, W1, b1, W2, b2, W3, b3, W4, b4, W5, b5,
           W6, b6, W7, b7, W8, b8, W9, b9, W10, b10):
    src = edge_index[0]
    dst = edge_index[1]
    src3 = src.reshape(NW, NCH, CH)
    dst3 = dst.reshape(NW, NCH, CH)
    srcA = src.reshape(NW, NCHA, CHA)
    dstA = dst.reshape(NW, NCHA, CHA)
    w3 = edge_weight.reshape(NW, NCH, CH)

    z1 = jnp.zeros((N,), f32)
    zD = {d: jnp.zeros((N, d), f32) for d in (16, 32)}

    deg2 = _sc_deg(dst3, w3, z1).reshape(NC, N)
    dinv, sn, t1 = _tc_first(deg2.T, x, W1)
    normf = _sc_norm(src, dst, edge_weight, dinv.reshape(N))

    def agg(t, d):
        k = {16: _agg16, 32: _agg32}[d]
        return k(t, srcA, dstA, normf, zD[d])

    a1 = agg(t1, 16)
    h2 = _tc_combine(a1, sn, t1, b1)
    a2 = agg(h2, 16)
    h3 = _tc_aggmm(a2, sn, h2, W2, b2)
    a3 = agg(h3, 32)
    h4 = _tc_aggmm(a3, sn, h3, W3, b3)
    a4a = agg(h4[:, :32], 32)
    a4b = agg(h4[:, 32:], 32)
    t5 = _tc_aggmm2_cat(a4a, a4b, sn, h4, W4, b4, W5)
    a5 = agg(t5, 32)
    h6 = _tc_combine(a5, sn, t5, b5)
    a6 = agg(h6, 32)
    t7 = _tc_aggmm2(a6, sn, h6, W6, b6, W7)
    a7 = agg(t7, 16)
    h8 = _tc_combine(a7, sn, t7, b7)
    a8 = agg(h8, 16)
    h9 = _tc_aggmm(a8, sn, h8, W8, b8)
    a9 = agg(h9, 16)
    t10 = _tc_aggmm2(a9, sn, h9, W9, b9, W10)
    a10 = _agg1(t10.reshape(N), src3, dst3, normf, z1).reshape(NC, N)
    out = _tc_final(a10.T, sn, t10, b10)
    return out.reshape(N)


# speedup vs baseline: 1.5774x; 1.0023x over previous
import functools

import jax
import jax.numpy as jnp
from jax import lax
from jax.experimental import pallas as pl
from jax.experimental.pallas import tpu as pltpu
from jax.experimental.pallas import tpu_sc as plsc

N = 10000
E = 320000
NC = 2
NS = 16
NW = NC * NS
EW = E // NW
CH = 80
NCH = EW // CH
CHA = 125
NCHA = EW // CHA
NBUF = 4
f32 = jnp.float32
i32 = jnp.int32


def _mesh():
    return plsc.VectorSubcoreMesh(core_axis_name="c", subcore_axis_name="s",
                                  num_cores=NC, num_subcores=NS)


@functools.partial(
    pl.kernel,
    out_type=jax.ShapeDtypeStruct((NC * N,), f32),
    mesh=_mesh(),
    compiler_params=pltpu.CompilerParams(needs_layout_passes=False, use_tc_tiling_on_sc=False),
    scratch_types=[pltpu.VMEM((NCH, CH), i32),
                   pltpu.VMEM((NCH, CH), f32),
                   pltpu.VMEM((1000,), f32),
                   pltpu.VMEM_SHARED((N,), f32)],
)
def _sc_deg(dst_h, w_h, z_h, out_h, dst_v, w_v, tmp_v, acc_sh):
    c = lax.axis_index("c")
    s = lax.axis_index("s")
    wid = c * NS + s

    @pl.when(s < 10)
    def _zero():
        sl = pl.ds(s * 1000, 1000)
        pltpu.sync_copy(z_h.at[sl], tmp_v)
        pltpu.sync_copy(tmp_v, acc_sh.at[sl])

    plsc.subcore_barrier()
    pltpu.sync_copy(dst_h.at[wid], dst_v)
    pltpu.sync_copy(w_h.at[wid], w_v)

    def chunk(k, car):
        pltpu.sync_copy(w_v.at[k], acc_sh.at[dst_v.at[k]], add=True)
        return car

    lax.fori_loop(0, NCH, chunk, 0)
    plsc.subcore_barrier()

    @pl.when(s < 10)
    def _wr():
        pltpu.sync_copy(acc_sh.at[pl.ds(s * 1000, 1000)], tmp_v)
        pltpu.sync_copy(tmp_v, out_h.at[pl.ds(c * N + s * 1000, 1000)])


@functools.partial(
    pl.kernel,
    out_type=jax.ShapeDtypeStruct((E,), f32),
    mesh=_mesh(),
    compiler_params=pltpu.CompilerParams(needs_layout_passes=False, use_tc_tiling_on_sc=False),
    scratch_types=[pltpu.VMEM((EW,), i32),
                   pltpu.VMEM((EW,), i32),
                   pltpu.VMEM((EW,), f32),
                   pltpu.VMEM((EW,), f32),
                   pltpu.VMEM((N,), f32)],
)
def _sc_norm(src_h, dst_h, w_h, dinv_h, out_h, src_v, dst_v, w_v, nv, dinv_v):
    c = lax.axis_index("c")
    s = lax.axis_index("s")
    wid = c * NS + s
    sl_e = pl.ds(wid * EW, EW)
    pltpu.sync_copy(src_h.at[sl_e], src_v)
    pltpu.sync_copy(dst_h.at[sl_e], dst_v)
    pltpu.sync_copy(w_h.at[sl_e], w_v)
    pltpu.sync_copy(dinv_h, dinv_v)

    def it(i, car):
        sl = pl.ds(i * 16, 16)
        gs = plsc.load_gather(dinv_v, [src_v[sl]])
        gd = plsc.load_gather(dinv_v, [dst_v[sl]])
        nv[sl] = gs * w_v[sl] * gd
        return car

    lax.fori_loop(0, EW // 16, it, 0)
    pltpu.sync_copy(nv, out_h.at[sl_e])


def _make_agg(D):
    scratch = [pltpu.VMEM((NCHA, CHA), i32),
               pltpu.VMEM((NCHA, CHA), i32),
               pltpu.VMEM((EW,), f32),
               pltpu.VMEM((1000, D), f32)]
    scratch += [pltpu.VMEM((CHA, D), f32) for _ in range(2 * NBUF)]
    scratch += [pltpu.SemaphoreType.DMA for _ in range(2 * NBUF)]
    scratch += [pltpu.VMEM_SHARED((N, D), f32)]
    spmem_tbl = D <= 16
    if spmem_tbl:
        scratch += [pltpu.VMEM_SHARED((N, D), f32)]

    @functools.partial(
        pl.kernel,
        out_type=jax.ShapeDtypeStruct((NC, N, D), f32),
        mesh=_mesh(),
        compiler_params=pltpu.CompilerParams(needs_layout_passes=False,
                                             use_tc_tiling_on_sc=False),
        scratch_types=scratch,
    )
    def agg(t_h, src_h, dst_h, norm_h, z_h, out_h, *sc):
        src_v, dst_v, norm_v, tmp_v = sc[0], sc[1], sc[2], sc[3]
        gbuf = sc[4:4 + NBUF]
        sbuf = sc[4 + NBUF:4 + 2 * NBUF]
        gsem = sc[4 + 2 * NBUF:4 + 3 * NBUF]
        ssem = sc[4 + 3 * NBUF:4 + 4 * NBUF]
        if spmem_tbl:
            acc_sh, tbl_sh = sc[-2], sc[-1]
        else:
            acc_sh, tbl_sh = sc[-1], t_h
        c = lax.axis_index("c")
        s = lax.axis_index("s")
        wid = c * NS + s

        @pl.when(s < 10)
        def _zero():
            sl = pl.ds(s * 1000, 1000)
            pltpu.sync_copy(z_h.at[sl], tmp_v)
            pltpu.sync_copy(tmp_v, acc_sh.at[sl])
            if spmem_tbl:
                pltpu.sync_copy(t_h.at[sl], tmp_v)
                pltpu.sync_copy(tmp_v, tbl_sh.at[sl])

        plsc.subcore_barrier()
        pltpu.sync_copy(src_h.at[wid], src_v)
        pltpu.sync_copy(dst_h.at[wid], dst_v)
        pltpu.sync_copy(norm_h.at[pl.ds(wid * EW, EW)], norm_v)

        for b in range(NBUF):
            pltpu.async_copy(tbl_sh.at[src_v.at[b]], gbuf[b], gsem[b])

        def round_(r, car):
            for b in range(NBUF):
                kk = r * NBUF + b
                pltpu.make_async_copy(
                    tbl_sh.at[src_v.at[kk]], gbuf[b], gsem[b]).wait()

                @pl.when(r > 0)
                def _drain():
                    pltpu.make_async_copy(
                        sbuf[b],
                        acc_sh.at[dst_v.at[lax.max(kk - NBUF, 0)]],
                        ssem[b]).wait()

                one = jnp.ones((16,), dtype=i32)

                def scale(i, idxv):
                    for u in range(25):
                        row = i * 25 + u
                        nsp = plsc.load_gather(norm_v, [idxv])
                        idxv = idxv + one
                        for j in range(D // 16):
                            sl = pl.ds(j * 16, 16)
                            sbuf[b][row, sl] = gbuf[b][row, sl] * nsp
                    return idxv

                lax.fori_loop(0, CHA // 25, scale,
                              jnp.full((16,), kk * CHA, dtype=i32))

                @pl.when(kk + NBUF < NCHA)
                def _regather():
                    pltpu.async_copy(
                        tbl_sh.at[src_v.at[kk + NBUF]], gbuf[b], gsem[b])

                pltpu.async_copy(
                    sbuf[b], acc_sh.at[dst_v.at[kk]], ssem[b], add=True)
            return car

        lax.fori_loop(0, NCHA // NBUF, round_, 0)
        for b in range(NBUF):
            pltpu.make_async_copy(
                sbuf[b], acc_sh.at[dst_v.at[NCHA - NBUF + b]],
                ssem[b]).wait()
        plsc.subcore_barrier()

        @pl.when(s < 10)
        def _wr():
            sl = pl.ds(s * 1000, 1000)
            pltpu.sync_copy(acc_sh.at[sl], tmp_v)
            pltpu.sync_copy(tmp_v, out_h.at[c, sl])

    return agg


_agg16 = _make_agg(16)
_agg32 = _make_agg(32)


@functools.partial(
    pl.kernel,
    out_type=jax.ShapeDtypeStruct((NC * N,), f32),
    mesh=_mesh(),
    compiler_params=pltpu.CompilerParams(needs_layout_passes=False, use_tc_tiling_on_sc=False),
    scratch_types=[pltpu.VMEM((NCH, CH), i32),
                   pltpu.VMEM((NCH, CH), i32),
                   pltpu.VMEM((EW,), f32),
                   pltpu.VMEM((CH,), f32),
                   pltpu.VMEM((1000,), f32),
                   pltpu.VMEM_SHARED((N,), f32),
                   pltpu.VMEM_SHARED((N,), f32)],
)
def _agg1(t_h, src_h, dst_h, norm_h, z_h, out_h,
          src_v, dst_v, norm_v, rows_v, tmp_v, acc_sh, tbl_sh):
    c = lax.axis_index("c")
    s = lax.axis_index("s")
    wid = c * NS + s

    @pl.when(s < 10)
    def _zero():
        sl = pl.ds(s * 1000, 1000)
        pltpu.sync_copy(z_h.at[sl], tmp_v)
        pltpu.sync_copy(tmp_v, acc_sh.at[sl])
        pltpu.sync_copy(t_h.at[sl], tmp_v)
        pltpu.sync_copy(tmp_v, tbl_sh.at[sl])

    plsc.subcore_barrier()
    pltpu.sync_copy(src_h.at[wid], src_v)
    pltpu.sync_copy(dst_h.at[wid], dst_v)
    pltpu.sync_copy(norm_h.at[pl.ds(wid * EW, EW)], norm_v)

    def chunk(k, car):
        pltpu.sync_copy(tbl_sh.at[src_v.at[k]], rows_v)
        for j in range(CH // 16):
            sl = pl.ds(j * 16, 16)
            rows_v[sl] = rows_v[sl] * norm_v[pl.ds(k * CH + j * 16, 16)]
        pltpu.sync_copy(rows_v, acc_sh.at[dst_v.at[k]], add=True)
        return car

    lax.fori_loop(0, NCH, chunk, 0)
    plsc.subcore_barrier()

    @pl.when(s < 10)
    def _wr():
        pltpu.sync_copy(acc_sh.at[pl.ds(s * 1000, 1000)], tmp_v)
        pltpu.sync_copy(tmp_v, out_h.at[pl.ds(c * N + s * 1000, 1000)])


def _tc(body, out_shapes):
    return pl.pallas_call(
        body,
        out_shape=[jax.ShapeDtypeStruct(s, f32) for s in out_shapes])


def _tc_first(deg2t, x, w1):
    def body(deg_ref, x_ref, w_ref, dinv_ref, sn_ref, t_ref):
        deg = deg_ref[:, 0:1] + deg_ref[:, 1:2] + 1.0
        di = lax.rsqrt(deg)
        dinv_ref[...] = di
        sn_ref[...] = di * di
        t_ref[...] = jnp.dot(x_ref[...], w_ref[...],
                             preferred_element_type=f32)
    return _tc(body, [(N, 1), (N, 1), (N, w1.shape[1])])(deg2t, x, w1)


def _tc_combine(a, sn, t, b):
    def body(a_ref, sn_ref, t_ref, b_ref, o_ref):
        h = a_ref[0] + a_ref[1] + sn_ref[...] * t_ref[...] + b_ref[...]
        o_ref[...] = jnp.maximum(h, 0.0)
    return _tc(body, [t.shape])(a, sn, t, b[None, :])[0]


def _tc_aggmm(a, sn, h, w, b):
    def body(a_ref, sn_ref, h_ref, w_ref, b_ref, o_ref):
        g = a_ref[0] + a_ref[1] + sn_ref[...] * h_ref[...]
        o_ref[...] = jnp.maximum(
            jnp.dot(g, w_ref[...], preferred_element_type=f32) + b_ref[...],
            0.0)
    return _tc(body, [(N, w.shape[1])])(a, sn, h, w, b[None, :])[0]


def _tc_aggmm2(a, sn, h, w, b, w2):
    def body(a_ref, sn_ref, h_ref, w_ref, b_ref, w2_ref, o_ref):
        g = a_ref[0] + a_ref[1] + sn_ref[...] * h_ref[...]
        hn = jnp.maximum(
            jnp.dot(g, w_ref[...], preferred_element_type=f32) + b_ref[...],
            0.0)
        o_ref[...] = jnp.dot(hn, w2_ref[...], preferred_element_type=f32)
    return _tc(body, [(N, w2.shape[1])])(a, sn, h, w, b[None, :], w2)[0]


def _tc_aggmm2_cat(aa, ab, sn, h, w, b, w2):
    def body(aa_ref, ab_ref, sn_ref, h_ref, w_ref, b_ref, w2_ref, o_ref):
        g = jnp.concatenate(
            [aa_ref[0] + aa_ref[1], ab_ref[0] + ab_ref[1]], axis=1)
        g = g + sn_ref[...] * h_ref[...]
        hn = jnp.maximum(
            jnp.dot(g, w_ref[...], preferred_element_type=f32) + b_ref[...],
            0.0)
        o_ref[...] = jnp.dot(hn, w2_ref[...], preferred_element_type=f32)
    return _tc(body, [(N, w2.shape[1])])(aa, ab, sn, h, w, b[None, :], w2)[0]


def _tc_final(a10t, sn, t10, b10):
    def body(a_ref, sn_ref, t_ref, b_ref, o_ref):
        o_ref[...] = jax.nn.sigmoid(
            a_ref[:, 0:1] + a_ref[:, 1:2]
            + sn_ref[...] * t_ref[...] + b_ref[...])
    return _tc(body, [(N, 1)])(a10t, sn, t10, b10[None, :])[0]


def kernel(x, edge_index, edge_weight, W1, b1, W2, b2, W3, b3, W4, b4, W5, b5,
           W6, b6, W7, b7, W8, b8, W9, b9, W10, b10):
    src = edge_index[0]
    dst = edge_index[1]
    src3 = src.reshape(NW, NCH, CH)
    dst3 = dst.reshape(NW, NCH, CH)
    srcA = src.reshape(NW, NCHA, CHA)
    dstA = dst.reshape(NW, NCHA, CHA)
    w3 = edge_weight.reshape(NW, NCH, CH)

    z1 = jnp.zeros((N,), f32)
    zD = {d: jnp.zeros((N, d), f32) for d in (16, 32)}

    deg2 = _sc_deg(dst3, w3, z1).reshape(NC, N)
    dinv, sn, t1 = _tc_first(deg2.T, x, W1)
    normf = _sc_norm(src, dst, edge_weight, dinv.reshape(N))

    def agg(t, d):
        k = {16: _agg16, 32: _agg32}[d]
        return k(t, srcA, dstA, normf, zD[d])

    a1 = agg(t1, 16)
    h2 = _tc_combine(a1, sn, t1, b1)
    a2 = agg(h2, 16)
    h3 = _tc_aggmm(a2, sn, h2, W2, b2)
    a3 = agg(h3, 32)
    h4 = _tc_aggmm(a3, sn, h3, W3, b3)
    a4a = agg(h4[:, :32], 32)
    a4b = agg(h4[:, 32:], 32)
    t5 = _tc_aggmm2_cat(a4a, a4b, sn, h4, W4, b4, W5)
    a5 = agg(t5, 32)
    h6 = _tc_combine(a5, sn, t5, b5)
    a6 = agg(h6, 32)
    t7 = _tc_aggmm2(a6, sn, h6, W6, b6, W7)
    a7 = agg(t7, 16)
    h8 = _tc_combine(a7, sn, t7, b7)
    a8 = agg(h8, 16)
    h9 = _tc_aggmm(a8, sn, h8, W8, b8)
    a9 = agg(h9, 16)
    t10 = _tc_aggmm2(a9, sn, h9, W9, b9, W10)
    a10 = _agg1(t10.reshape(N), src3, dst3, normf, z1).reshape(NC, N)
    out = _tc_final(a10.T, sn, t10, b10)
    return out.reshape(N)
